# baseline (speedup 1.0000x reference)
import functools

import jax
import jax.numpy as jnp
from jax import lax
from jax.experimental import pallas as pl
from jax.experimental.pallas import tpu as pltpu
from jax.experimental.pallas import tpu_sc as plsc

NUM_CORES = 2
NUM_SUBCORES = 16
NUM_WORKERS = NUM_CORES * NUM_SUBCORES
LANES = 16


def _build(batch, maxlen, vocab, embed):
    assert batch % NUM_WORKERS == 0
    assert embed % LANES == 0 and (maxlen * embed) % 8 == 0
    seq_per_w = batch // NUM_WORKERS
    rows = maxlen
    mesh = plsc.VectorSubcoreMesh(
        core_axis_name="c", subcore_axis_name="s",
        num_cores=NUM_CORES, num_subcores=NUM_SUBCORES)

    gather_slices = []
    off = 0
    while off < rows:
        n = min(128, rows - off)
        gather_slices.append((off, n))
        off += n

    @functools.partial(
        pl.kernel,
        out_type=jax.ShapeDtypeStruct((batch, maxlen, embed), jnp.float32),
        mesh=mesh,
        scratch_types=[
            pltpu.VMEM((rows,), jnp.int32),
            pltpu.VMEM((rows, embed), jnp.float32),
            pltpu.VMEM_SHARED((maxlen, embed), jnp.float32),
            pltpu.SemaphoreType.DMA,
        ],
        compiler_params=pltpu.CompilerParams(use_tc_tiling_on_sc=False),
    )
    def k(x_hbm, tok_hbm, pos_hbm, out_hbm, idx_v, buf_v, pos_s, sem):
        cid = lax.axis_index("c")
        sid = lax.axis_index("s")
        wid = sid * NUM_CORES + cid

        @pl.when(sid == 0)
        def _fill():
            pltpu.sync_copy(pos_hbm, pos_s)

        plsc.subcore_barrier()

        @pl.loop(0, seq_per_w)
        def _seq(g):
            seq = wid * seq_per_w + g
            pltpu.sync_copy(x_hbm.at[seq], idx_v)
            pltpu.sync_copy(pos_s, buf_v)
            cps = []
            for off, n in gather_slices:
                cps.append(
                    pltpu.async_copy(
                        tok_hbm.at[idx_v.at[pl.ds(off, n)]],
                        buf_v.at[pl.ds(off, n)],
                        sem,
                        add=True,
                    )
                )
            for cp in cps:
                cp.wait()
            pltpu.sync_copy(buf_v, out_hbm.at[seq])

    return k


def kernel(x, token_table, pos_table):
    batch, maxlen = x.shape
    vocab, embed = token_table.shape
    tok2 = jnp.pad(token_table, ((0, 0), (0, 128 - embed))).reshape(
        vocab * 128 // embed, embed)
    x2 = (x * 2).astype(jnp.int32)
    half = maxlen // 2
    k = _build(batch, half, vocab * 128 // embed, embed)
    a = k(x2[:, :half], tok2, pos_table[:half])
    b = k(x2[:, half:], tok2, pos_table[half:])
    return jnp.concatenate([a, b], axis=1)


# speedup vs baseline: 1.2147x; 1.2147x over previous
import functools

import jax
import jax.numpy as jnp
from jax import lax
from jax.experimental import pallas as pl
from jax.experimental.pallas import tpu as pltpu
from jax.experimental.pallas import tpu_sc as plsc

NUM_CORES = 2
NUM_SUBCORES = 16
NUM_WORKERS = NUM_CORES * NUM_SUBCORES
LANES = 16


def _build(batch, maxlen, vocab, embed):
    assert batch % NUM_WORKERS == 0
    assert embed % LANES == 0 and (maxlen * embed) % 8 == 0
    seq_per_w = batch // NUM_WORKERS
    rows = maxlen
    mesh = plsc.VectorSubcoreMesh(
        core_axis_name="c", subcore_axis_name="s",
        num_cores=NUM_CORES, num_subcores=NUM_SUBCORES)

    gather_slices = []
    off = 0
    while off < rows:
        n = min(128, rows - off)
        gather_slices.append((off, n))
        off += n

    @functools.partial(
        pl.kernel,
        out_type=jax.ShapeDtypeStruct((batch, maxlen, embed), jnp.float32),
        mesh=mesh,
        scratch_types=[
            pltpu.VMEM((rows,), jnp.int32),
            pltpu.VMEM((rows, embed), jnp.float32),
            pltpu.VMEM_SHARED((maxlen, embed), jnp.float32),
            pltpu.SemaphoreType.DMA,
        ],
        compiler_params=pltpu.CompilerParams(use_tc_tiling_on_sc=False),
    )
    def k(x_hbm, tok_hbm, pos_hbm, out_hbm, idx_v, buf_v, pos_s, sem):
        cid = lax.axis_index("c")
        sid = lax.axis_index("s")
        wid = sid * NUM_CORES + cid

        @pl.when(sid == 0)
        def _fill():
            pltpu.sync_copy(pos_hbm, pos_s)

        plsc.subcore_barrier()

        @pl.loop(0, seq_per_w)
        def _seq(g):
            seq = wid * seq_per_w + g
            pltpu.sync_copy(x_hbm.at[seq], idx_v)
            pltpu.sync_copy(pos_s, buf_v)
            cps = []
            for off, n in gather_slices:
                cps.append(
                    pltpu.async_copy(
                        tok_hbm.at[idx_v.at[pl.ds(off, n)]],
                        buf_v.at[pl.ds(off, n)],
                        sem,
                        add=True,
                    )
                )
            for cp in cps:
                cp.wait()
            pltpu.sync_copy(buf_v, out_hbm.at[seq])

    return k


def kernel(x, token_table, pos_table):
    batch, maxlen = x.shape
    vocab, embed = token_table.shape
    tok2 = jnp.pad(token_table, ((0, 0), (0, 128 - embed))).reshape(
        vocab * 128 // embed, embed)
    x2 = (x * 2).astype(jnp.int32)
    k = _build(batch, maxlen, vocab * 128 // embed, embed)
    return k(x2, tok2, pos_table)
